# baseline (speedup 1.0000x reference)
import functools

import jax
import jax.numpy as jnp
from jax import lax
from jax.experimental import pallas as pl
from jax.experimental.pallas import tpu as pltpu
from jax.experimental.pallas import tpu_sc as plsc

B = 16384
D = 64
NC = 2
NS = 16
NW = NC * NS
BPW = B // NW
CHUNK = 128
NCHUNK = BPW // CHUNK

MLP_BLK = 2048


def _gather_body(table_hbm, idx_hbm, out_hbm, idx_v, rows_a, rows_b, sem_g):
    wid = lax.axis_index("s") * NC + lax.axis_index("c")
    base = wid * BPW
    cbase = wid * NCHUNK
    pltpu.sync_copy(idx_hbm.at[pl.ds(cbase, NCHUNK)], idx_v)
    bufs = (rows_a, rows_b)
    pltpu.async_copy(table_hbm.at[idx_v.at[0]], bufs[0], sem_g)
    pltpu.async_copy(table_hbm.at[idx_v.at[1]], bufs[1], sem_g)
    for c in range(NCHUNK):
        pltpu.make_async_copy(
            table_hbm.at[idx_v.at[c]], bufs[c % 2], sem_g).wait()
        pltpu.sync_copy(
            bufs[c % 2], out_hbm.at[pl.ds(base + c * CHUNK, CHUNK)])
        if c + 2 < NCHUNK:
            pltpu.async_copy(
                table_hbm.at[idx_v.at[c + 2]], bufs[c % 2], sem_g)


@functools.cache
def _gather_kernel():
    mesh = plsc.VectorSubcoreMesh(
        core_axis_name="c", subcore_axis_name="s",
        num_cores=NC, num_subcores=NS)
    return pl.kernel(
        _gather_body,
        out_type=jax.ShapeDtypeStruct((B, D), jnp.float32),
        mesh=mesh,
        compiler_params=pltpu.CompilerParams(use_tc_tiling_on_sc=False),
        scratch_types=[
            pltpu.VMEM((NCHUNK, CHUNK), jnp.int32),
            pltpu.VMEM((CHUNK, D), jnp.float32),
            pltpu.VMEM((CHUNK, D), jnp.float32),
            pltpu.SemaphoreType.DMA,
        ],
    )


def _mlp_body(x_ref, g_ref, bt_ref, w1_ref, b1_ref, w2_ref, b2_ref, o_ref):
    x = x_ref[...]
    mu = jnp.mean(x, axis=-1, keepdims=True)
    xc = x - mu
    var = jnp.mean(xc * xc, axis=-1, keepdims=True)
    xn = xc * lax.rsqrt(var + 1e-5) * g_ref[...] + bt_ref[...]
    h = jnp.dot(xn, w1_ref[...], preferred_element_type=jnp.float32) + b1_ref[...]
    h = h * jax.nn.sigmoid(h)
    o_ref[...] = lax.dot_general(
        w2_ref[...], h, (((0,), (1,)), ((), ())),
        preferred_element_type=jnp.float32) + b2_ref[...]


def _mlp(x, gamma, beta, W1, b1, W2, b2):
    full = lambda i: (0, 0)
    return pl.pallas_call(
        _mlp_body,
        grid=(B // MLP_BLK,),
        in_specs=[
            pl.BlockSpec((MLP_BLK, D), lambda i: (i, 0)),
            pl.BlockSpec((1, D), full),
            pl.BlockSpec((1, D), full),
            pl.BlockSpec((D, D), full),
            pl.BlockSpec((1, D), full),
            pl.BlockSpec((D, D), full),
            pl.BlockSpec((D, 1), full),
        ],
        out_specs=pl.BlockSpec((D, MLP_BLK), lambda i: (0, i)),
        out_shape=jax.ShapeDtypeStruct((D, B), jnp.float32),
    )(x, gamma.reshape(1, D), beta.reshape(1, D), W1,
      b1.reshape(1, D), W2, b2.reshape(D, 1))


def kernel(bands, band_emb, gamma, beta, W1, b1, W2, b2):
    idx = bands.astype(jnp.int32).reshape(NW * NCHUNK, CHUNK)
    gathered = _gather_kernel()(band_emb, idx)
    out_t = _mlp(gathered, gamma, beta, W1, b1, W2, b2)
    return out_t.T
